# baseline (speedup 1.0000x reference)
import functools

import jax
import jax.numpy as jnp
from jax.experimental import pallas as pl
from jax.experimental.pallas import tpu as pltpu

_TN = 512


def _split3_bf16(v):
    h = v.astype(jnp.bfloat16)
    r = v - h.astype(jnp.float32)
    m = r.astype(jnp.bfloat16)
    l = (r - m.astype(jnp.float32)).astype(jnp.bfloat16)
    return h, m, l


def _chamfer_tile_kernel(a_ref, b_ref, xmin_ref, ymin_ref):
    i = pl.program_id(1)
    d = jnp.dot(
        a_ref[0], b_ref[0], preferred_element_type=jnp.float32
    )
    xmin_ref[0, 0, 0, :] = jnp.min(d, axis=1)
    ymin_tile = jnp.min(d, axis=0, keepdims=True)[None]

    @pl.when(i == 0)
    def _init():
        ymin_ref[...] = ymin_tile

    @pl.when(i > 0)
    def _acc():
        ymin_ref[...] = jnp.minimum(ymin_ref[...], ymin_tile)


@functools.partial(jax.jit, static_argnames=())
def _chamfer(predict, target):
    B, N, _ = predict.shape
    _, M, _ = target.shape
    f32 = jnp.float32
    bf16 = jnp.bfloat16

    xx = jnp.sum(predict * predict, axis=-1, keepdims=True)
    xxh, xxm, xxl = _split3_bf16(xx)
    ones_a = jnp.ones((B, N, 3), dtype=bf16)
    pad_a = jnp.zeros((B, N, 7), dtype=bf16)
    amat = jnp.concatenate(
        [predict.astype(bf16), xxh, xxm, xxl, ones_a, pad_a], axis=-1
    )

    ty = target.transpose(0, 2, 1)
    yy = jnp.sum(ty * ty, axis=1, keepdims=True)
    yyh, yym, yyl = _split3_bf16(yy)
    ones_b = jnp.ones((B, 3, M), dtype=bf16)
    pad_b = jnp.zeros((B, 7, M), dtype=bf16)
    bmat = jnp.concatenate(
        [(-2.0 * ty).astype(bf16), ones_b, yyh, yym, yyl, pad_b], axis=1
    )

    nb = N // _TN
    x_near, y_near = pl.pallas_call(
        _chamfer_tile_kernel,
        grid=(B, nb),
        in_specs=[
            pl.BlockSpec((1, _TN, 16), lambda b, i: (b, i, 0)),
            pl.BlockSpec((1, 16, M), lambda b, i: (b, 0, 0)),
        ],
        out_specs=[
            pl.BlockSpec((1, 1, 1, _TN), lambda b, i: (b, i, 0, 0)),
            pl.BlockSpec((1, 1, M), lambda b, i: (b, 0, 0)),
        ],
        out_shape=[
            jax.ShapeDtypeStruct((B, nb, 1, _TN), f32),
            jax.ShapeDtypeStruct((B, 1, M), f32),
        ],
        compiler_params=pltpu.CompilerParams(
            dimension_semantics=("parallel", "arbitrary"),
        ),
    )(amat, bmat)
    return x_near.mean() + y_near.mean()


def kernel(predict, target):
    return _chamfer(predict, target)
